# baseline (speedup 1.0000x reference)
import functools

import jax
import jax.numpy as jnp
import numpy as np
from jax.experimental import pallas as pl

K = 3
IN_N = 32
OUT_N = 32
SQ = 4
D = 16
TEMP = 0.75
SINKHORN_ITER = 7
LN_EPS = 1e-5
R = 4


def _body(x_ref, kc_ref, g_ref, wnk_ref, gam_ref, bet_ref, o_ref):
    P = R * 32

    xloc = x_ref[0].reshape(3 * 32, R + 2, 32, 16)
    z3 = jax.lax.dot_general(
        xloc, kc_ref[...],
        dimension_numbers=(((3,), (1,)), ((0,), (0,))),
        preferred_element_type=jnp.float32)

    parts = {}
    for ki in range(K):
        for kj in range(K):
            sl = z3[kj * 32:(kj + 1) * 32, ki:ki + R, :, ki * 16:(ki + 1) * 16]
            parts[(ki, kj)] = sl.reshape(32, 1, P, 16)
    vt = jnp.concatenate(
        [parts[(ki, kj)] for ki in range(K) for kj in range(K)],
        axis=1).reshape(288, P, 16)

    q0 = jnp.mean(vt, axis=0)
    qw = jax.lax.dot_general(
        q0, g_ref[...], dimension_numbers=(((1,), (0,)), ((), ())),
        preferred_element_type=jnp.float32).reshape(P, OUT_N, 16)
    logits = jax.lax.dot_general(
        qw, vt, dimension_numbers=(((2,), (2,)), ((0,), (1,))),
        preferred_element_type=jnp.float32)

    zl = logits * (1.0 / (np.sqrt(float(D)) * TEMP))
    zl = zl - jnp.max(zl, axis=(1, 2), keepdims=True)
    attn = jnp.exp(zl)
    for _ in range(SINKHORN_ITER):
        attn = attn * (1.0 / jnp.sum(attn, axis=2, keepdims=True))
        attn = attn * (1.0 / jnp.sum(attn, axis=1, keepdims=True))

    u = jax.lax.dot_general(
        attn, vt, dimension_numbers=(((2,), (0,)), ((0,), (1,))),
        preferred_element_type=jnp.float32)
    nxt = jnp.einsum('pme,mef->pmf', u, wnk_ref[...],
                     preferred_element_type=jnp.float32)

    mean = jnp.mean(nxt, axis=-1, keepdims=True)
    var = jnp.mean((nxt - mean) ** 2, axis=-1, keepdims=True)
    y = (nxt - mean) * jax.lax.rsqrt(var + LN_EPS)
    y = y * gam_ref[...].reshape(1, 1, 16) + bet_ref[...].reshape(1, 1, 16)

    o_ref[0] = y.reshape(R, 32, OUT_N, 16).transpose(2, 0, 1, 3)


@functools.partial(jax.jit, static_argnames=("interpret",))
def _run(x, w_current, w_next, ln_gamma, ln_beta, interpret=False):
    B, N, H, W, _ = x.shape
    HB = H // R
    xp = jnp.pad(x, ((0, 0), (0, 0), (1, 1), (1, 1), (0, 0)))
    xs3 = jnp.stack([xp[:, :, :, kj:kj + W, :] for kj in range(K)],
                    axis=1)
    row_idx = (jnp.arange(HB)[:, None] * R + jnp.arange(R + 2)[None, :]).reshape(-1)
    xwin = jnp.take(xs3, row_idx, axis=3)

    wc = jnp.transpose(w_current, (2, 0, 1, 3, 4)).reshape(IN_N, K * K, SQ, SQ)
    eye = jnp.eye(SQ, dtype=jnp.float32)
    kr = jnp.einsum('ax,ctjb->ctajxb', eye, wc)
    kr = kr.reshape(IN_N, K, K, 16, 16)
    kc3 = jnp.stack([kr[:, :, kj].transpose(0, 2, 1, 3).reshape(IN_N, 16, K * 16)
                     for kj in range(K)], axis=0).reshape(K * IN_N, 16, K * 16)
    g = jnp.einsum('mbk,mck->mbc', w_next, w_next)
    gt = jnp.einsum('xa,mcb->xcmab', eye, g).reshape(16, OUT_N * 16)
    wnk = jnp.einsum('ax,mbc->mabxc', eye, w_next).reshape(OUT_N, 16, 16)
    gam = ln_gamma.reshape(1, 16)
    bet = ln_beta.reshape(1, 16)

    grid = (B, HB)
    out = pl.pallas_call(
        _body,
        grid=grid,
        in_specs=[
            pl.BlockSpec((1, 3, N, R + 2, W, 16),
                         lambda b, h: (b, 0, 0, h, 0, 0)),
            pl.BlockSpec((K * IN_N, 16, K * 16), lambda b, h: (0, 0, 0)),
            pl.BlockSpec((16, OUT_N * 16), lambda b, h: (0, 0)),
            pl.BlockSpec((OUT_N, 16, 16), lambda b, h: (0, 0, 0)),
            pl.BlockSpec((1, 16), lambda b, h: (0, 0)),
            pl.BlockSpec((1, 16), lambda b, h: (0, 0)),
        ],
        out_specs=pl.BlockSpec((1, OUT_N, R, W, 16), lambda b, h: (b, 0, h, 0, 0)),
        out_shape=jax.ShapeDtypeStruct((B, OUT_N, H, W, 16), jnp.float32),
        interpret=interpret,
    )(xwin, kc3, gt, wnk, gam, bet)
    return out


def kernel(input, w_current, w_next, ln_gamma, ln_beta):
    return _run(input, w_current, w_next, ln_gamma, ln_beta)


# speedup vs baseline: 1.1864x; 1.1864x over previous
import functools

import jax
import jax.numpy as jnp
import numpy as np
from jax.experimental import pallas as pl

K = 3
IN_N = 32
OUT_N = 32
SQ = 4
D = 16
TEMP = 0.75
SINKHORN_ITER = 7
LN_EPS = 1e-5
R = 4


def _body(x_ref, kc_ref, g_ref, wnk_ref, gam_ref, bet_ref, o_ref):
    P = R * 32

    xloc = x_ref[0].reshape(3 * 32, R + 2, 32, 16)
    z3 = jax.lax.dot_general(
        xloc, kc_ref[...],
        dimension_numbers=(((3,), (1,)), ((0,), (0,))),
        preferred_element_type=jnp.float32)

    parts = {}
    for ki in range(K):
        for kj in range(K):
            sl = z3[kj * 32:(kj + 1) * 32, ki:ki + R, :, ki * 16:(ki + 1) * 16]
            parts[(ki, kj)] = sl.reshape(32, 1, P, 16)
    vt = jnp.concatenate(
        [parts[(ki, kj)] for ki in range(K) for kj in range(K)],
        axis=1).reshape(288, P, 16)
    vtb = jnp.transpose(vt, (1, 2, 0))

    q0 = jnp.mean(vtb, axis=2)
    qw = jax.lax.dot_general(
        q0, g_ref[...], dimension_numbers=(((1,), (0,)), ((), ())),
        preferred_element_type=jnp.float32).reshape(P, OUT_N, 16)
    logits = jax.lax.dot_general(
        qw, vtb, dimension_numbers=(((2,), (1,)), ((0,), (0,))),
        preferred_element_type=jnp.float32)

    zl = logits * (1.0 / (np.sqrt(float(D)) * TEMP))
    zl = zl - jnp.max(zl, axis=(1, 2), keepdims=True)
    p0 = jnp.exp(zl)
    r = 1.0 / jnp.sum(p0, axis=2)
    c = 1.0 / jax.lax.dot_general(
        r, p0, dimension_numbers=(((1,), (1,)), ((0,), (0,))),
        preferred_element_type=jnp.float32)
    for _ in range(SINKHORN_ITER - 1):
        r = 1.0 / jax.lax.dot_general(
            p0, c, dimension_numbers=(((2,), (1,)), ((0,), (0,))),
            preferred_element_type=jnp.float32)
        c = 1.0 / jax.lax.dot_general(
            r, p0, dimension_numbers=(((1,), (1,)), ((0,), (0,))),
            preferred_element_type=jnp.float32)

    vtc = vtb * c[:, None, :]
    u = jax.lax.dot_general(
        p0, vtc, dimension_numbers=(((2,), (2,)), ((0,), (0,))),
        preferred_element_type=jnp.float32) * r[:, :, None]
    nxt = jnp.einsum('pme,mef->pmf', u, wnk_ref[...],
                     preferred_element_type=jnp.float32)

    mean = jnp.mean(nxt, axis=-1, keepdims=True)
    var = jnp.mean((nxt - mean) ** 2, axis=-1, keepdims=True)
    y = (nxt - mean) * jax.lax.rsqrt(var + LN_EPS)
    y = y * gam_ref[...].reshape(1, 1, 16) + bet_ref[...].reshape(1, 1, 16)

    o_ref[0] = y.reshape(R, 32, OUT_N, 16).transpose(2, 0, 1, 3)


@functools.partial(jax.jit, static_argnames=("interpret",))
def _run(x, w_current, w_next, ln_gamma, ln_beta, interpret=False):
    B, N, H, W, _ = x.shape
    HB = H // R
    xp = jnp.pad(x, ((0, 0), (0, 0), (1, 1), (1, 1), (0, 0)))
    xs3 = jnp.stack([xp[:, :, :, kj:kj + W, :] for kj in range(K)],
                    axis=1)
    xwin = jnp.stack(
        [jax.lax.dynamic_slice_in_dim(xs3, hb * R, R + 2, axis=3)
         for hb in range(HB)], axis=1)
    xwin = xwin.reshape(B * HB, 3, N, R + 2, W, 16)

    wc = jnp.transpose(w_current, (2, 0, 1, 3, 4)).reshape(IN_N, K * K, SQ, SQ)
    eye = jnp.eye(SQ, dtype=jnp.float32)
    kr = jnp.einsum('ax,ctjb->ctajxb', eye, wc)
    kr = kr.reshape(IN_N, K, K, 16, 16)
    kc3 = jnp.stack([kr[:, :, kj].transpose(0, 2, 1, 3).reshape(IN_N, 16, K * 16)
                     for kj in range(K)], axis=0).reshape(K * IN_N, 16, K * 16)
    g = jnp.einsum('mbk,mck->mbc', w_next, w_next)
    gt = jnp.einsum('xa,mcb->xcmab', eye, g).reshape(16, OUT_N * 16)
    wnk = jnp.einsum('ax,mbc->mabxc', eye, w_next).reshape(OUT_N, 16, 16)
    gam = ln_gamma.reshape(1, 16)
    bet = ln_beta.reshape(1, 16)

    grid = (B, HB)
    out = pl.pallas_call(
        _body,
        grid=grid,
        in_specs=[
            pl.BlockSpec((1, 3, N, R + 2, W, 16),
                         lambda b, h: (b * HB + h, 0, 0, 0, 0, 0)),
            pl.BlockSpec((K * IN_N, 16, K * 16), lambda b, h: (0, 0, 0)),
            pl.BlockSpec((16, OUT_N * 16), lambda b, h: (0, 0)),
            pl.BlockSpec((OUT_N, 16, 16), lambda b, h: (0, 0, 0)),
            pl.BlockSpec((1, 16), lambda b, h: (0, 0)),
            pl.BlockSpec((1, 16), lambda b, h: (0, 0)),
        ],
        out_specs=pl.BlockSpec((1, OUT_N, R, W, 16), lambda b, h: (b, 0, h, 0, 0)),
        out_shape=jax.ShapeDtypeStruct((B, OUT_N, H, W, 16), jnp.float32),
        interpret=interpret,
    )(xwin, kc3, gt, wnk, gam, bet)
    return out


def kernel(input, w_current, w_next, ln_gamma, ln_beta):
    return _run(input, w_current, w_next, ln_gamma, ln_beta)
